# baseline (speedup 1.0000x reference)
import functools

import jax
import jax.numpy as jnp
from jax import lax
from jax.experimental import pallas as pl
from jax.experimental.pallas import tpu as pltpu
from jax.experimental.pallas import tpu_sc as plsc

_SQRT_2_OVER_PI = 0.7978845608028654
_EPS = 1e-5

NC = 2
NS = 16
L = 16


def _head_body(x_ref, w_ref, b_ref, g_ref, be_ref, o_ref):
    xb = x_ref[...]
    h = lax.dot_general(xb, w_ref[...], (((1,), (1,)), ((), ())),
                        preferred_element_type=jnp.float32)
    h = h + b_ref[...]
    h = 0.5 * h * (1.0 + jnp.tanh(_SQRT_2_OVER_PI * (h + 0.044715 * h * h * h)))
    mean = jnp.mean(h, axis=-1, keepdims=True)
    var = jnp.mean(jnp.square(h - mean), axis=-1, keepdims=True)
    o_ref[...] = g_ref[...] * (h - mean) * lax.rsqrt(var + _EPS) + be_ref[...]


def _dense_head(x, W, b, gamma, beta):
    B, INP = x.shape
    H = W.shape[0]
    blk = 512
    return pl.pallas_call(
        _head_body,
        grid=(B // blk,),
        in_specs=[
            pl.BlockSpec((blk, INP), lambda i: (i, 0)),
            pl.BlockSpec((H, INP), lambda i: (0, 0)),
            pl.BlockSpec((1, H), lambda i: (0, 0)),
            pl.BlockSpec((1, H), lambda i: (0, 0)),
            pl.BlockSpec((1, H), lambda i: (0, 0)),
        ],
        out_specs=pl.BlockSpec((blk, H), lambda i: (i, 0)),
        out_shape=jax.ShapeDtypeStruct((B, H), jnp.float32),
    )(x, W, b.reshape(1, H), gamma.reshape(1, H), beta.reshape(1, H))


def _make_sc_dot(B, C, H, V):
    NW = NC * NS
    ROWS = B // NW
    assert C >= L and C % 8 == 0
    offs = list(range(0, C - L + 1, L))
    if offs[-1] + L < C:
        offs.append(C - L)

    mesh = plsc.VectorSubcoreMesh(core_axis_name="c", subcore_axis_name="s")

    @functools.partial(
        pl.kernel,
        out_type=jax.ShapeDtypeStruct((B * C,), jnp.float32),
        mesh=mesh,
        compiler_params=pltpu.CompilerParams(needs_layout_passes=False,
                                             use_tc_tiling_on_sc=False),
        scratch_types=[
            pltpu.VMEM((ROWS, H), jnp.float32),
            pltpu.VMEM((ROWS * C,), jnp.int32),
            pltpu.VMEM((C, H), jnp.float32),
            pltpu.VMEM((C, H), jnp.float32),
            pltpu.VMEM((C,), jnp.float32),
            pltpu.VMEM((C,), jnp.float32),
            pltpu.VMEM((ROWS * C,), jnp.float32),
            pltpu.SemaphoreType.DMA,
            pltpu.SemaphoreType.DMA,
            pltpu.SemaphoreType.DMA,
            pltpu.SemaphoreType.DMA,
        ],
    )
    def sc_dot(h_hbm, cand_hbm, table_hbm, bias_hbm, out_hbm,
               h_v, cand_v, emb0, emb1, bias0, bias1, out_v,
               se0, se1, sb0, sb1):
        cid = lax.axis_index("c")
        sid = lax.axis_index("s")
        wid = sid * NC + cid
        base = wid * ROWS

        pltpu.sync_copy(h_hbm.at[pl.ds(base, ROWS)], h_v)
        pltpu.sync_copy(cand_hbm.at[pl.ds(base * C, ROWS * C)], cand_v)

        iota = lax.iota(jnp.int32, L)

        def start_row(r, emb_b, bias_b, sem_e, sem_b):
            idx = cand_v.at[pl.ds(r * C, C)]
            pltpu.async_copy(table_hbm.at[idx], emb_b, sem_e)
            pltpu.async_copy(bias_hbm.at[idx], bias_b, sem_b)

        def wait_row(emb_b, bias_b, sem_e, sem_b):
            pltpu.make_async_copy(table_hbm.at[cand_v.at[pl.ds(0, C)]],
                                  emb_b, sem_e).wait()
            pltpu.make_async_copy(bias_hbm.at[cand_v.at[pl.ds(0, C)]],
                                  bias_b, sem_b).wait()

        def compute_row(r, emb_b, bias_b):
            out_base = r * C
            hv = [h_v[r, pl.ds(k * L, L)] for k in range(H // L)]
            for o in offs:
                cidx = o + iota
                acc = bias_b[pl.ds(o, L)]
                for d in range(H):
                    col = jnp.full((L,), d, dtype=jnp.int32)
                    e = plsc.load_gather(emb_b, [cidx, col])
                    acc = acc + e * hv[d // L][d % L]
                out_v[pl.ds(out_base + o, L)] = acc

        def body(i, carry):
            r0 = 2 * i
            wait_row(emb0, bias0, se0, sb0)
            compute_row(r0, emb0, bias0)

            @pl.when(r0 + 2 < ROWS)
            def _():
                start_row(r0 + 2, emb0, bias0, se0, sb0)

            wait_row(emb1, bias1, se1, sb1)
            compute_row(r0 + 1, emb1, bias1)

            @pl.when(r0 + 3 < ROWS)
            def _():
                start_row(r0 + 3, emb1, bias1, se1, sb1)

            return carry

        start_row(0, emb0, bias0, se0, sb0)
        start_row(1, emb1, bias1, se1, sb1)
        lax.fori_loop(0, ROWS // 2, body, 0)

        pltpu.sync_copy(out_v, out_hbm.at[pl.ds(base * C, ROWS * C)])

    return sc_dot


def kernel(x, candidates, W, b, gamma, beta, table, bias):
    B, C = candidates.shape
    V, H = table.shape
    h = _dense_head(x, W, b, gamma, beta)
    cand_flat = candidates.astype(jnp.int32).reshape(-1)
    bias_flat = bias.reshape(-1)
    sc_dot = _make_sc_dot(B, C, H, V)
    logits = sc_dot(h, cand_flat, table, bias_flat)
    return logits.reshape(B, C)


# speedup vs baseline: 2.0133x; 2.0133x over previous
import functools

import jax
import jax.numpy as jnp
from jax import lax
from jax.experimental import pallas as pl
from jax.experimental.pallas import tpu as pltpu
from jax.experimental.pallas import tpu_sc as plsc

_SQRT_2_OVER_PI = 0.7978845608028654
_EPS = 1e-5

NC = 2
NS = 16
L = 16


def _head_body(x_ref, w_ref, b_ref, g_ref, be_ref, o_ref):
    xb = x_ref[...]
    h = lax.dot_general(xb, w_ref[...], (((1,), (1,)), ((), ())),
                        preferred_element_type=jnp.float32)
    h = h + b_ref[...]
    h = 0.5 * h * (1.0 + jnp.tanh(_SQRT_2_OVER_PI * (h + 0.044715 * h * h * h)))
    mean = jnp.mean(h, axis=-1, keepdims=True)
    var = jnp.mean(jnp.square(h - mean), axis=-1, keepdims=True)
    o_ref[...] = g_ref[...] * (h - mean) * lax.rsqrt(var + _EPS) + be_ref[...]


def _dense_head(x, W, b, gamma, beta):
    B, INP = x.shape
    H = W.shape[0]
    blk = 512
    return pl.pallas_call(
        _head_body,
        grid=(B // blk,),
        in_specs=[
            pl.BlockSpec((blk, INP), lambda i: (i, 0)),
            pl.BlockSpec((H, INP), lambda i: (0, 0)),
            pl.BlockSpec((1, H), lambda i: (0, 0)),
            pl.BlockSpec((1, H), lambda i: (0, 0)),
            pl.BlockSpec((1, H), lambda i: (0, 0)),
        ],
        out_specs=pl.BlockSpec((blk, H), lambda i: (i, 0)),
        out_shape=jax.ShapeDtypeStruct((B, H), jnp.float32),
    )(x, W, b.reshape(1, H), gamma.reshape(1, H), beta.reshape(1, H))


def _make_sc_dot(B, C, H, V):
    NW = NC * NS
    ROWS = B // NW
    assert C >= L and C % 8 == 0
    offs = list(range(0, C - L + 1, L))
    if offs[-1] + L < C:
        offs.append(C - L)
    NBUF = 4
    assert ROWS % NBUF == 0

    mesh = plsc.VectorSubcoreMesh(core_axis_name="c", subcore_axis_name="s")

    @functools.partial(
        pl.kernel,
        out_type=jax.ShapeDtypeStruct((B * C,), jnp.float32),
        mesh=mesh,
        compiler_params=pltpu.CompilerParams(needs_layout_passes=False,
                                             use_tc_tiling_on_sc=False),
        scratch_types=[
            pltpu.VMEM((ROWS, H), jnp.float32),
            pltpu.VMEM((ROWS * C,), jnp.int32),
            [pltpu.VMEM((C, H), jnp.float32) for _ in range(NBUF)],
            [pltpu.VMEM((C,), jnp.float32) for _ in range(NBUF)],
            pltpu.VMEM((H + L,), jnp.float32),
            pltpu.VMEM((ROWS * C,), jnp.float32),
            [pltpu.SemaphoreType.DMA for _ in range(NBUF)],
            [pltpu.SemaphoreType.DMA for _ in range(NBUF)],
        ],
    )
    def sc_dot(h_hbm, cand_hbm, table_hbm, bias_hbm, out_hbm,
               h_v, cand_v, embs, biases, hext, out_v, sems_e, sems_b):
        cid = lax.axis_index("c")
        sid = lax.axis_index("s")
        wid = sid * NC + cid
        base = wid * ROWS

        pltpu.sync_copy(h_hbm.at[pl.ds(base, ROWS)], h_v)
        pltpu.sync_copy(cand_hbm.at[pl.ds(base * C, ROWS * C)], cand_v)

        iota = lax.iota(jnp.int32, L)
        rows_g = [o + iota for o in offs]

        def start_row(r, j):
            idx = cand_v.at[pl.ds(r * C, C)]
            pltpu.async_copy(table_hbm.at[idx], embs[j], sems_e[j])
            pltpu.async_copy(bias_hbm.at[idx], biases[j], sems_b[j])

        def wait_row(j):
            idx = cand_v.at[pl.ds(0, C)]
            pltpu.make_async_copy(table_hbm.at[idx], embs[j], sems_e[j]).wait()
            pltpu.make_async_copy(bias_hbm.at[idx], biases[j], sems_b[j]).wait()

        def compute_row(r, j):
            for k in range(H // L):
                hext[pl.ds(k * L, L)] = h_v[r, pl.ds(k * L, L)]
            hext[pl.ds(H, L)] = h_v[r, pl.ds(0, L)]
            emb_b, bias_b = embs[j], biases[j]

            def iloop(i, accs):
                t = iota + i
                col = lax.bitwise_and(t, H - 1)
                hwin = plsc.load_gather(hext, [t])
                return tuple(
                    acc + plsc.load_gather(emb_b, [rg, col]) * hwin
                    for acc, rg in zip(accs, rows_g)
                )

            accs = lax.fori_loop(
                0, H, iloop,
                tuple(bias_b[pl.ds(o, L)] for o in offs))
            out_base = r * C
            for acc, o in zip(accs, offs):
                out_v[pl.ds(out_base + o, L)] = acc

        def body(i, carry):
            for j in range(NBUF):
                wait_row(j)
                compute_row(NBUF * i + j, j)

                @pl.when(i < ROWS // NBUF - 1)
                def _():
                    start_row(NBUF * i + j + NBUF, j)

            return carry

        for j in range(NBUF):
            start_row(j, j)
        lax.fori_loop(0, ROWS // NBUF, body, 0)

        pltpu.sync_copy(out_v, out_hbm.at[pl.ds(base * C, ROWS * C)])

    return sc_dot


def kernel(x, candidates, W, b, gamma, beta, table, bias):
    B, C = candidates.shape
    V, H = table.shape
    h = _dense_head(x, W, b, gamma, beta)
    cand_flat = candidates.astype(jnp.int32).reshape(-1)
    bias_flat = bias.reshape(-1)
    sc_dot = _make_sc_dot(B, C, H, V)
    logits = sc_dot(h, cand_flat, table, bias_flat)
    return logits.reshape(B, C)
